# Initial kernel scaffold; baseline (speedup 1.0000x reference)
import jax
import jax.numpy as jnp
from jax.experimental import pallas as pl


def kernel(H, J, sigma_i, r_tab, sigma_r_tab, w_b_tab):
    raise NotImplementedError("write your pallas kernel here")


# speedup vs baseline: 3.2556x; 3.2556x over previous
import functools

import jax
import jax.numpy as jnp
from jax import lax
from jax.experimental import pallas as pl
from jax.experimental.pallas import tpu as pltpu


def _stream_kernel(w_ref, j_ref, out_ref, *, half):
    c = pl.program_id(0)
    j = pl.program_id(1)
    r = c * half + j

    @pl.when(j == 0)
    def _():
        out_ref[...] = jnp.zeros_like(out_ref)

    jb = j_ref[...]
    lane = lax.broadcasted_iota(jnp.int32, jb.shape, 1)
    masked = jnp.where(lane == r, 0.0, jb * w_ref[...])
    msum = jnp.sum(masked, axis=1, keepdims=True)
    ssq = jnp.sum(jb * jb)
    col = jnp.concatenate([msum, jnp.broadcast_to(ssq, (1, 1))], axis=0)

    olane = lax.broadcasted_iota(jnp.int32, out_ref.shape, 2)
    out_ref[...] = jnp.where(olane == j, col[None], out_ref[...])


def _finish_kernel(gp_ref, h_ref, sel_ref, p_ref, q_ref, wb_ref, out_ref,
                   *, qq, lambda_h, lambda_j):
    gsum = jnp.concatenate([gp_ref[0], gp_ref[1]], axis=1)
    h = h_ref[...]

    g = lax.dot_general(sel_ref[...], gsum, (((1,), (0,)), ((), ())),
                        precision=lax.Precision.HIGHEST,
                        preferred_element_type=jnp.float32)
    row = lax.broadcasted_iota(jnp.int32, gsum.shape, 0)
    ssq_j = jnp.sum(jnp.where(row == qq, gsum, 0.0))
    reg = lambda_h * jnp.sum(h * h) + lambda_j * ssq_j

    e_all = h + g
    e2 = lax.dot_general(e_all, p_ref[...], (((1,), (0,)), ((), ())),
                         precision=lax.Precision.HIGHEST,
                         preferred_element_type=jnp.float32)
    m = jnp.max(e2, axis=0, keepdims=True)
    lse = m + jnp.log(jnp.sum(jnp.exp(e2 - m), axis=0, keepdims=True))
    e_sel = jnp.sum(e2 * q_ref[...], axis=0, keepdims=True)
    out_ref[...] = (lse - e_sel) * wb_ref[...] + reg


def kernel(H, J, sigma_i, r_tab, sigma_r_tab, w_b_tab):
    q, L = H.shape
    qq = q * q
    R = int(r_tab.shape[0])
    half = L // 2
    lambda_h = 0.01
    lambda_j = 0.01

    Hf = jnp.asarray(H, jnp.float32)
    Jf = jnp.asarray(J, jnp.float32)
    sig = jnp.asarray(sigma_i, jnp.int32)

    brow = (jnp.arange(qq, dtype=jnp.int32) % q)[:, None]
    w_mask = (sig[None, :] == brow).astype(jnp.float32)
    sel = (jnp.arange(qq, dtype=jnp.int32)[None, :] // q
           == jnp.arange(q, dtype=jnp.int32)[:, None]).astype(jnp.float32)
    sel = jnp.pad(sel, ((0, 0), (0, 1)))
    p_gather = (jnp.arange(L, dtype=jnp.int32)[:, None]
                == jnp.asarray(r_tab, jnp.int32)[None, :]).astype(jnp.float32)
    q_gather = (jnp.arange(q, dtype=jnp.int32)[:, None]
                == jnp.asarray(sigma_r_tab, jnp.int32)[None, :]).astype(jnp.float32)
    wb = jnp.asarray(w_b_tab, jnp.float32).reshape(1, R)

    gp = pl.pallas_call(
        functools.partial(_stream_kernel, half=half),
        out_shape=jax.ShapeDtypeStruct((2, qq + 1, half), jnp.float32),
        grid=(2, half),
        in_specs=[
            pl.BlockSpec((qq, L), lambda c, j: (0, 0)),
            pl.BlockSpec((qq, L), lambda c, j: (0, c * half + j)),
        ],
        out_specs=pl.BlockSpec((1, qq + 1, half), lambda c, j: (c, 0, 0)),
        compiler_params=pltpu.CompilerParams(
            dimension_semantics=("parallel", "arbitrary")),
    )(w_mask, Jf)

    out = pl.pallas_call(
        functools.partial(_finish_kernel, qq=qq,
                          lambda_h=lambda_h, lambda_j=lambda_j),
        out_shape=jax.ShapeDtypeStruct((1, R), jnp.float32),
    )(gp, Hf, sel, p_gather, q_gather, wb)
    return out.reshape(R)
